# baseline (speedup 1.0000x reference)
import functools

import numpy as np
import jax
import jax.numpy as jnp
from jax import lax
from jax.experimental import pallas as pl
from jax.experimental.pallas import tpu as pltpu
from jax.experimental.pallas import tpu_sc as plsc

_RES = (16, 64, 256)
_NCH = 100
_DIM = 64
_B = 16384
_NC, _NS, _L = 2, 16, 16
_NW = _NC * _NS
_P = 10
_CHUNK = 256
_NCHUNKS = _B // _CHUNK
_ZROWS = 16
_BOFF = (0, 16, 80)
_STRIPE = _B // _NS


def _make_partition():
    pairs = []
    for i, r in enumerate(_RES):
        for c in range(_NCH):
            pairs.append((i, c, r))
    pairs.sort(key=lambda t: -t[2])
    njobs = [0] * _NW
    rows = [0] * _NW
    assign = [[] for _ in range(_NW)]
    for (i, c, r) in pairs:
        cand = [t for t in range(_NW) if njobs[t] < _P]
        t = min(cand, key=lambda t: (njobs[t], rows[t]))
        assign[t].append((i, c, r))
        njobs[t] += 1
        rows[t] += r
    maxrows = _ZROWS + max(rows)
    maxrows = (maxrows + 63) // 64 * 64

    w_base = [0]
    for r in _RES[:-1]:
        w_base.append(w_base[-1] + _NCH * (r + 1))

    row_map = np.zeros((_NW, maxrows), dtype=np.int32)
    meta_i = np.zeros((_NW, 128), dtype=np.int32)
    meta_f = np.full((_NW, 128), float(_RES[0] - 1), dtype=np.float32)
    for t in range(_NW):
        assert len(assign[t]) >= 1
        off = _ZROWS
        for p, (i, c, r) in enumerate(assign[t]):
            g0 = w_base[i] + c * (r + 1)
            row_map[t, off:off + r] = np.arange(g0, g0 + r, dtype=np.int32)
            meta_i[t, 1 + p] = c
            meta_i[t, 17 + p] = off
            meta_i[t, 33 + p] = _BOFF[i]
            meta_f[t, 1 + p] = float(r - 1)
            off += r
        for p in range(len(assign[t]), _P):
            meta_i[t, 1 + p] = 0
            meta_i[t, 17 + p] = 0
            meta_i[t, 33 + p] = 0
            meta_f[t, 1 + p] = float(_RES[0] - 1)
    return row_map, meta_i, meta_f, maxrows


_ROW_MAP, _META_I, _META_F, _MAXROWS = _make_partition()
_MI_W = 128


def _sc_body(x_hbm, wcat_hbm, rmap_hbm, bnds_hbm, mi_hbm, mf_hbm, zer_hbm,
             out_hbm,
             xbuf, tbl, bnd_v, mi_v, mf_v, accb, idxb, ridx, sem,
             acc_a, acc_b):
    cid = lax.axis_index("c")
    sid = lax.axis_index("s")
    wid = sid * _NC + cid
    iota = lax.iota(jnp.int32, _L)

    pltpu.sync_copy(bnds_hbm, bnd_v)
    pltpu.sync_copy(mi_hbm.at[wid], mi_v)
    pltpu.sync_copy(mf_hbm.at[wid], mf_v)

    zero = jnp.zeros((_L,), jnp.float32)

    for j in range(_MAXROWS // 64):
        pltpu.sync_copy(rmap_hbm.at[wid, pl.ds(j * 64, 64)], ridx)
        pltpu.async_copy(wcat_hbm.at[ridx],
                         tbl.at[pl.ds(j * 64, 64)], sem).wait()
    for rr in range(_ZROWS):
        for k in range(_DIM // _L):
            tbl[rr, pl.ds(k * _L, _L)] = zero
    zb = _CHUNK // _NS
    pltpu.sync_copy(zer_hbm.at[pl.ds(0, zb)], acc_a.at[pl.ds(sid * zb, zb)])
    pltpu.sync_copy(zer_hbm.at[pl.ds(0, zb)], acc_b.at[pl.ds(sid * zb, zb)])
    for v in range(_CHUNK // _L):
        idxb[pl.ds(v * _L, _L)] = v * _L + iota
    plsc.subcore_barrier()

    def sv_body(sv, _):
        srow = sv * _L + iota
        rowbase = []
        for p in range(_P):
            pv = jnp.full((_L,), 1 + p, jnp.int32)
            ch_v = plsc.load_gather(mi_v, [pv])
            ro_v = plsc.load_gather(mi_v, [pv + 16])
            bo_v = plsc.load_gather(mi_v, [pv + 32])
            sc_v = plsc.load_gather(mf_v, [pv])
            xv = plsc.load_gather(xbuf, [srow, ch_v])
            y = xv * sc_v
            t = y.astype(jnp.int32)
            g = t + jnp.where(t.astype(jnp.float32) < y, 1, 0)
            gb = bo_v + g
            bg = plsc.load_gather(bnd_v, [gb])
            bgm1 = plsc.load_gather(bnd_v, [jnp.maximum(gb - 1, bo_v)])
            dec = jnp.logical_and(g > 0, bgm1 >= xv)
            g = g - jnp.where(dec, 1, 0) + jnp.where(bg < xv, 1, 0)
            rowbase.append(ro_v + g)
        for d in range(_DIM):
            dv = jnp.full((_L,), d, jnp.int32)
            vals = [plsc.load_gather(tbl, [rowbase[p], dv]) for p in range(_P)]
            while len(vals) > 1:
                vals = [a + b for a, b in zip(vals[::2], vals[1::2])] + (
                    [vals[-1]] if len(vals) % 2 else [])
            plsc.store_scatter(accb, [srow, dv], vals[0])
        return _

    def chunk_body(ck, _):
        slot = lax.rem(ck, 2)
        pltpu.sync_copy(x_hbm.at[pl.ds(ck * _CHUNK, _CHUNK), :], xbuf)
        lax.fori_loop(0, _CHUNK // _L, sv_body, None)
        @pl.when(slot == 0)
        def _add_a():
            pltpu.sync_copy(accb, acc_a.at[idxb], add=True)
        @pl.when(slot == 1)
        def _add_b():
            pltpu.sync_copy(accb, acc_b.at[idxb], add=True)
        plsc.subcore_barrier()
        @pl.when(sid == lax.rem(ck, _NS))
        def _flush():
            @pl.when(slot == 0)
            def _fa():
                pltpu.sync_copy(acc_a, out_hbm.at[cid, pl.ds(ck * _CHUNK, _CHUNK)])
                pltpu.sync_copy(zer_hbm, acc_a)
            @pl.when(slot == 1)
            def _fb():
                pltpu.sync_copy(acc_b, out_hbm.at[cid, pl.ds(ck * _CHUNK, _CHUNK)])
                pltpu.sync_copy(zer_hbm, acc_b)
        return _

    lax.fori_loop(0, _NCHUNKS, chunk_body, None)


def _tc_add_body(a_ref, b_ref, o_ref):
    o_ref[...] = a_ref[...] + b_ref[...]


@jax.jit
def kernel(x, W_0, W_1, W_2):
    w_cat = jnp.concatenate([W_0, W_1, W_2], axis=0)
    rmap = jnp.asarray(_ROW_MAP)
    bnds = jnp.concatenate(
        [jnp.linspace(0.0, 1.0, r) for r in _RES]).astype(jnp.float32)
    mi = jnp.asarray(_META_I)
    mf = jnp.asarray(_META_F)

    mesh = plsc.VectorSubcoreMesh(core_axis_name="c", subcore_axis_name="s")
    sc = pl.kernel(
        _sc_body,
        out_type=jax.ShapeDtypeStruct((_NC, _B, _DIM), jnp.float32),
        mesh=mesh,
        compiler_params=pltpu.CompilerParams(
            needs_layout_passes=False, use_tc_tiling_on_sc=False),
        scratch_types=[
            pltpu.VMEM((_CHUNK, 128), jnp.float32),
            pltpu.VMEM((_MAXROWS, _DIM), jnp.float32),
            pltpu.VMEM((sum(_RES),), jnp.float32),
            pltpu.VMEM((_MI_W,), jnp.int32),
            pltpu.VMEM((128,), jnp.float32),
            pltpu.VMEM((_CHUNK, _DIM), jnp.float32),
            pltpu.VMEM((_CHUNK,), jnp.int32),
            pltpu.VMEM((64,), jnp.int32),
            pltpu.SemaphoreType.DMA,
            pltpu.VMEM_SHARED((_CHUNK, _DIM), jnp.float32),
            pltpu.VMEM_SHARED((_CHUNK, _DIM), jnp.float32),
        ],
    )
    x_pad = jnp.pad(x, ((0, 0), (0, 128 - _NCH)))
    zer = jnp.zeros((_CHUNK, _DIM), jnp.float32)
    partial = sc(x_pad, w_cat, rmap, bnds, mi, mf, zer)

    blk = 1024
    out = pl.pallas_call(
        _tc_add_body,
        grid=(_B // blk,),
        in_specs=[pl.BlockSpec((blk, _DIM), lambda i: (i, 0))] * 2,
        out_specs=pl.BlockSpec((blk, _DIM), lambda i: (i, 0)),
        out_shape=jax.ShapeDtypeStruct((_B, _DIM), jnp.float32),
    )(partial[0], partial[1])
    return out


# speedup vs baseline: 1.0313x; 1.0313x over previous
import functools

import numpy as np
import jax
import jax.numpy as jnp
from jax import lax
from jax.experimental import pallas as pl
from jax.experimental.pallas import tpu as pltpu
from jax.experimental.pallas import tpu_sc as plsc

_RES = (16, 64, 256)
_NCH = 100
_DIM = 64
_B = 16384
_NC, _NS, _L = 2, 16, 16
_NW = _NC * _NS
_P = 10
_CHUNK = 256
_NCHUNKS = _B // _CHUNK
_ZROWS = 16
_BOFF = (0, 16, 80)
_STRIPE = _B // _NS


def _make_partition():
    pairs = []
    for i, r in enumerate(_RES):
        for c in range(_NCH):
            pairs.append((i, c, r))
    pairs.sort(key=lambda t: -t[2])
    njobs = [0] * _NW
    rows = [0] * _NW
    assign = [[] for _ in range(_NW)]
    for (i, c, r) in pairs:
        cand = [t for t in range(_NW) if njobs[t] < _P]
        t = min(cand, key=lambda t: (njobs[t], rows[t]))
        assign[t].append((i, c, r))
        njobs[t] += 1
        rows[t] += r
    maxrows = _ZROWS + max(rows)
    maxrows = (maxrows + 63) // 64 * 64

    w_base = [0]
    for r in _RES[:-1]:
        w_base.append(w_base[-1] + _NCH * (r + 1))

    row_map = np.zeros((_NW, maxrows), dtype=np.int32)
    meta_i = np.zeros((_NW, 128), dtype=np.int32)
    meta_f = np.full((_NW, 128), float(_RES[0] - 1), dtype=np.float32)
    for t in range(_NW):
        assert len(assign[t]) >= 1
        off = _ZROWS
        for p, (i, c, r) in enumerate(assign[t]):
            g0 = w_base[i] + c * (r + 1)
            row_map[t, off:off + r] = np.arange(g0, g0 + r, dtype=np.int32)
            meta_i[t, 1 + p] = c
            meta_i[t, 17 + p] = off
            meta_i[t, 33 + p] = _BOFF[i]
            meta_f[t, 1 + p] = float(r - 1)
            off += r
        for p in range(len(assign[t]), _P):
            meta_i[t, 1 + p] = 0
            meta_i[t, 17 + p] = 0
            meta_i[t, 33 + p] = 0
            meta_f[t, 1 + p] = float(_RES[0] - 1)
    return row_map, meta_i, meta_f, maxrows


_ROW_MAP, _META_I, _META_F, _MAXROWS = _make_partition()
_MI_W = 128


def _sc_body(x_hbm, wcat_hbm, rmap_hbm, bnds_hbm, mi_hbm, mf_hbm, zer_hbm,
             out_hbm,
             xbuf, tbl, bnd_v, mi_v, mf_v, accb, idxb, ridx, sem,
             acc_a, acc_b):
    cid = lax.axis_index("c")
    sid = lax.axis_index("s")
    wid = sid * _NC + cid
    iota = lax.iota(jnp.int32, _L)

    pltpu.sync_copy(bnds_hbm, bnd_v)
    pltpu.sync_copy(mi_hbm.at[wid], mi_v)
    pltpu.sync_copy(mf_hbm.at[wid], mf_v)

    zero = jnp.zeros((_L,), jnp.float32)

    for j in range(_MAXROWS // 64):
        pltpu.sync_copy(rmap_hbm.at[wid, pl.ds(j * 64, 64)], ridx)
        pltpu.async_copy(wcat_hbm.at[ridx],
                         tbl.at[pl.ds(j * 64, 64)], sem).wait()
    for rr in range(_ZROWS):
        for k in range(_DIM // _L):
            tbl[rr, pl.ds(k * _L, _L)] = zero
    zb = _CHUNK // _NS
    pltpu.sync_copy(zer_hbm.at[pl.ds(0, zb)], acc_a.at[pl.ds(sid * zb, zb)])
    pltpu.sync_copy(zer_hbm.at[pl.ds(0, zb)], acc_b.at[pl.ds(sid * zb, zb)])
    for v in range(_CHUNK // _L):
        idxb[pl.ds(v * _L, _L)] = v * _L + iota
    plsc.subcore_barrier()

    def sv_body(sv):
        srow = sv * _L + iota
        rowbase = []
        for p in range(_P):
            pv = jnp.full((_L,), 1 + p, jnp.int32)
            ch_v = plsc.load_gather(mi_v, [pv])
            ro_v = plsc.load_gather(mi_v, [pv + 16])
            bo_v = plsc.load_gather(mi_v, [pv + 32])
            sc_v = plsc.load_gather(mf_v, [pv])
            xv = plsc.load_gather(xbuf, [srow, ch_v])
            y = xv * sc_v
            t = y.astype(jnp.int32)
            g = t + jnp.where(t.astype(jnp.float32) < y, 1, 0)
            gb = bo_v + g
            bg = plsc.load_gather(bnd_v, [gb])
            bgm1 = plsc.load_gather(bnd_v, [jnp.maximum(gb - 1, bo_v)])
            dec = jnp.logical_and(g > 0, bgm1 >= xv)
            g = g - jnp.where(dec, 1, 0) + jnp.where(bg < xv, 1, 0)
            rowbase.append(ro_v + g)
        for d in range(_DIM):
            dv = jnp.full((_L,), d, jnp.int32)
            vals = [plsc.load_gather(tbl, [rowbase[p], dv]) for p in range(_P)]
            while len(vals) > 1:
                vals = [a + b for a, b in zip(vals[::2], vals[1::2])] + (
                    [vals[-1]] if len(vals) % 2 else [])
            plsc.store_scatter(accb, [srow, dv], vals[0])

    def chunk_body(ck, _):
        slot = lax.rem(ck, 2)
        pltpu.sync_copy(x_hbm.at[pl.ds(ck * _CHUNK, _CHUNK), :], xbuf)
        plsc.parallel_loop(0, _CHUNK // _L, 1, unroll=2)(sv_body)
        @pl.when(slot == 0)
        def _add_a():
            pltpu.sync_copy(accb, acc_a.at[idxb], add=True)
        @pl.when(slot == 1)
        def _add_b():
            pltpu.sync_copy(accb, acc_b.at[idxb], add=True)
        plsc.subcore_barrier()
        @pl.when(sid == lax.rem(ck, _NS))
        def _flush():
            @pl.when(slot == 0)
            def _fa():
                pltpu.sync_copy(acc_a, out_hbm.at[cid, pl.ds(ck * _CHUNK, _CHUNK)])
                pltpu.sync_copy(zer_hbm, acc_a)
            @pl.when(slot == 1)
            def _fb():
                pltpu.sync_copy(acc_b, out_hbm.at[cid, pl.ds(ck * _CHUNK, _CHUNK)])
                pltpu.sync_copy(zer_hbm, acc_b)
        return _

    lax.fori_loop(0, _NCHUNKS, chunk_body, None)


def _tc_add_body(a_ref, b_ref, o_ref):
    o_ref[...] = a_ref[...] + b_ref[...]


@jax.jit
def kernel(x, W_0, W_1, W_2):
    w_cat = jnp.concatenate([W_0, W_1, W_2], axis=0)
    rmap = jnp.asarray(_ROW_MAP)
    bnds = jnp.concatenate(
        [jnp.linspace(0.0, 1.0, r) for r in _RES]).astype(jnp.float32)
    mi = jnp.asarray(_META_I)
    mf = jnp.asarray(_META_F)

    mesh = plsc.VectorSubcoreMesh(core_axis_name="c", subcore_axis_name="s")
    sc = pl.kernel(
        _sc_body,
        out_type=jax.ShapeDtypeStruct((_NC, _B, _DIM), jnp.float32),
        mesh=mesh,
        compiler_params=pltpu.CompilerParams(
            needs_layout_passes=False, use_tc_tiling_on_sc=False),
        scratch_types=[
            pltpu.VMEM((_CHUNK, 128), jnp.float32),
            pltpu.VMEM((_MAXROWS, _DIM), jnp.float32),
            pltpu.VMEM((sum(_RES),), jnp.float32),
            pltpu.VMEM((_MI_W,), jnp.int32),
            pltpu.VMEM((128,), jnp.float32),
            pltpu.VMEM((_CHUNK, _DIM), jnp.float32),
            pltpu.VMEM((_CHUNK,), jnp.int32),
            pltpu.VMEM((64,), jnp.int32),
            pltpu.SemaphoreType.DMA,
            pltpu.VMEM_SHARED((_CHUNK, _DIM), jnp.float32),
            pltpu.VMEM_SHARED((_CHUNK, _DIM), jnp.float32),
        ],
    )
    x_pad = jnp.pad(x, ((0, 0), (0, 128 - _NCH)))
    zer = jnp.zeros((_CHUNK, _DIM), jnp.float32)
    partial = sc(x_pad, w_cat, rmap, bnds, mi, mf, zer)

    blk = 1024
    out = pl.pallas_call(
        _tc_add_body,
        grid=(_B // blk,),
        in_specs=[pl.BlockSpec((blk, _DIM), lambda i: (i, 0))] * 2,
        out_specs=pl.BlockSpec((blk, _DIM), lambda i: (i, 0)),
        out_shape=jax.ShapeDtypeStruct((_B, _DIM), jnp.float32),
    )(partial[0], partial[1])
    return out
